# Initial kernel scaffold; baseline (speedup 1.0000x reference)
import jax
import jax.numpy as jnp
from jax.experimental import pallas as pl


def kernel(joint1_embedding, joint2_embedding, operation):
    raise NotImplementedError("write your pallas kernel here")


# speedup vs baseline: 1.1476x; 1.1476x over previous
import jax
import jax.numpy as jnp
from jax.experimental import pallas as pl
from jax.experimental.pallas import tpu as pltpu

_OP_EQ, _OP_LT, _OP_GT = 0, 1, 2
_ALPHA, _BETA = 1.2, 0.7
_B, _D = 8192, 2048
_BM = 512
_NB = _B // _BM


def _loss_body(op_full_ref, op_blk_ref, j1_ref, j2_ref, out_ref, acc_ref,
               stats_ref):
    i = pl.program_id(0)

    @pl.when(i == 0)
    def _init_stats():
        op_full = op_full_ref[0, :]
        stats_ref[0] = jnp.sum((op_full == _OP_EQ).astype(jnp.float32))
        stats_ref[1] = jnp.sum((op_full == _OP_LT).astype(jnp.float32))
        stats_ref[2] = jnp.sum((op_full == _OP_GT).astype(jnp.float32))

    j1 = j1_ref[...]
    j2 = j2_ref[...]
    d = j1 - j2
    sd = jnp.sum(d * d, axis=1)
    s1 = jnp.sum(j1 * j1, axis=1)
    s2 = jnp.sum(j2 * j2, axis=1)
    op = op_blk_ref[0, 0, :]
    eq = (op == _OP_EQ).astype(jnp.float32)
    dn = jnp.sqrt(s1) - jnp.sqrt(s2)
    relu_lt = jnp.maximum(dn, 0.0)
    relu_gt = jnp.maximum(-dn, 0.0)

    @pl.when(i == 0)
    def _init_acc():
        acc_ref[0, :] = eq * sd
        acc_ref[1, :] = relu_lt
        acc_ref[2, :] = relu_gt

    @pl.when(i > 0)
    def _accumulate():
        acc_ref[0, :] += eq * sd
        acc_ref[1, :] += relu_lt
        acc_ref[2, :] += relu_gt

    @pl.when(i == _NB - 1)
    def _finalize():
        eq_cnt = stats_ref[0]
        has_lt = (stats_ref[1] > 0.0).astype(jnp.float32)
        has_gt = (stats_ref[2] > 0.0).astype(jnp.float32)
        eq_loss = jnp.sum(acc_ref[0, :]) / jnp.maximum(eq_cnt * _D, 1.0)
        lt_loss = jnp.sum(acc_ref[1, :]) * (1.0 / _B)
        gt_loss = jnp.sum(acc_ref[2, :]) * (1.0 / _B)
        out_ref[0, 0] = (_ALPHA * eq_loss
                         + _BETA * (has_lt * lt_loss + has_gt * gt_loss))


def kernel(joint1_embedding, joint2_embedding, operation):
    op_row = operation.reshape(1, _B)
    op_blocks = operation.reshape(_NB, 1, _BM)
    out = pl.pallas_call(
        _loss_body,
        grid=(_NB,),
        in_specs=[
            pl.BlockSpec((1, _B), lambda i: (0, 0)),
            pl.BlockSpec((1, 1, _BM), lambda i: (i, 0, 0)),
            pl.BlockSpec((_BM, _D), lambda i: (i, 0)),
            pl.BlockSpec((_BM, _D), lambda i: (i, 0)),
        ],
        out_specs=pl.BlockSpec(memory_space=pltpu.SMEM),
        out_shape=jax.ShapeDtypeStruct((1, 1), jnp.float32),
        scratch_shapes=[
            pltpu.VMEM((3, _BM), jnp.float32),
            pltpu.SMEM((3,), jnp.float32),
        ],
    )(op_row, op_blocks, joint1_embedding, joint2_embedding)
    return out[0, 0]
